# Initial kernel scaffold; baseline (speedup 1.0000x reference)
import jax
import jax.numpy as jnp
from jax.experimental import pallas as pl


def kernel(x, hyperedge_index, bias):
    raise NotImplementedError("write your pallas kernel here")


# speedup vs baseline: 5.0401x; 5.0401x over previous
import functools

import jax
import jax.numpy as jnp
from jax import lax
from jax.experimental import pallas as pl
from jax.experimental.pallas import tpu as pltpu
from jax.experimental.pallas import tpu_sc as plsc

N_NODES = 10000
N_EDGES = 2000
D_FEAT = 256
N_INC = 160000
Q = 64
NS = 16
PER_TILE = N_INC // NS
K = 80
NB = PER_TILE // K
E_SLAB = N_EDGES // NS
N_SLAB = N_NODES // NS
CHUNK = 125

_mesh = plsc.VectorSubcoreMesh(
    core_axis_name="c", subcore_axis_name="s", num_cores=2, num_subcores=NS
)


@functools.partial(
    pl.kernel,
    out_type=jax.ShapeDtypeStruct((4 * N_NODES * Q,), jnp.float32),
    mesh=_mesh,
    compiler_params=pltpu.CompilerParams(use_tc_tiling_on_sc=False),
    scratch_types=[
        pltpu.VMEM((K, Q), jnp.float32),
        pltpu.VMEM((K,), jnp.int32),
        pltpu.VMEM((K,), jnp.int32),
        pltpu.VMEM((K, 16), jnp.float32),
        pltpu.VMEM((CHUNK, Q), jnp.float32),
        pltpu.VMEM((CHUNK, 16), jnp.float32),
        pltpu.VMEM((CHUNK * Q,), jnp.float32),
        pltpu.VMEM((2 * Q,), jnp.float32),
        pltpu.VMEM_SHARED((N_EDGES, Q), jnp.float32),
        pltpu.VMEM_SHARED((N_NODES, Q), jnp.float32),
        pltpu.VMEM_SHARED((N_EDGES, 16), jnp.float32),
        pltpu.VMEM_SHARED((N_NODES, 16), jnp.float32),
        pltpu.SemaphoreType.DMA,
    ],
)
def _hyper_gcn_sc(
    x4, nidx_hbm, eidx_hbm, bias_hbm, out_hbm,
    rows_v, nidx_v, eidx_v, ones_v, wbuf, cbuf, obuf, bias_v,
    e_acc, n_acc, e_cnt, n_cnt, sem,
):
    cid = lax.axis_index("c")
    sid = lax.axis_index("s")
    zeros16 = jnp.zeros((16,), jnp.float32)
    ones16 = jnp.ones((16,), jnp.float32)
    ebase = sid * E_SLAB

    def _fill_ones(r, carry):
        ones_v[r, :] = ones16
        return carry

    lax.fori_loop(0, K, _fill_ones, 0)

    def _zero_cnt_row(r, carry):
        cbuf[r, :] = zeros16
        return carry

    lax.fori_loop(0, CHUNK, _zero_cnt_row, 0)

    pltpu.sync_copy(bias_hbm.at[pl.ds(cid * 2 * Q, 2 * Q)], bias_v)

    for rnd in range(2):
        q = 2 * cid + rnd

        def _zero_row(r, carry):
            for c in range(Q // 16):
                wbuf[r, pl.ds(c * 16, 16)] = zeros16
            return carry

        lax.fori_loop(0, CHUNK, _zero_row, 0)
        pltpu.sync_copy(wbuf, e_acc.at[pl.ds(ebase, CHUNK)])
        if rnd == 0:
            pltpu.sync_copy(cbuf, e_cnt.at[pl.ds(ebase, CHUNK)])

        def _zero_nodes(j, carry):
            nb = sid * N_SLAB + j * CHUNK
            pltpu.sync_copy(wbuf, n_acc.at[pl.ds(nb, CHUNK)])
            if rnd == 0:
                pltpu.sync_copy(cbuf, n_cnt.at[pl.ds(nb, CHUNK)])
            return carry

        lax.fori_loop(0, N_SLAB // CHUNK, _zero_nodes, 0)
        plsc.subcore_barrier()

        def _stage1(b, carry):
            base = sid * PER_TILE + b * K
            pltpu.sync_copy(nidx_hbm.at[pl.ds(base, K)], nidx_v)
            pltpu.sync_copy(eidx_hbm.at[pl.ds(base, K)], eidx_v)
            if rnd == 0:
                pltpu.sync_copy(ones_v, n_cnt.at[nidx_v], add=True)
                pltpu.sync_copy(ones_v, e_cnt.at[eidx_v], add=True)
            for i in range(K // 16):
                nidx_v[pl.ds(i * 16, 16)] = nidx_v[pl.ds(i * 16, 16)] * 4 + q
            pltpu.async_copy(x4.at[nidx_v], rows_v, sem).wait()
            pltpu.sync_copy(rows_v, e_acc.at[eidx_v], add=True)
            return carry

        lax.fori_loop(0, NB, _stage1, 0)
        plsc.subcore_barrier()

        pltpu.sync_copy(e_acc.at[pl.ds(ebase, E_SLAB)], wbuf.at[pl.ds(0, E_SLAB)])
        pltpu.sync_copy(e_cnt.at[pl.ds(ebase, E_SLAB)], cbuf.at[pl.ds(0, E_SLAB)])

        def _scale_edge(r, carry):
            cnt = cbuf[r, :]
            rs = jnp.where(cnt > 0.0, 1.0 / cnt, 0.0)
            for c in range(Q // 16):
                wbuf[r, pl.ds(c * 16, 16)] = wbuf[r, pl.ds(c * 16, 16)] * rs
            return carry

        lax.fori_loop(0, E_SLAB, _scale_edge, 0)
        pltpu.sync_copy(wbuf.at[pl.ds(0, E_SLAB)], e_acc.at[pl.ds(ebase, E_SLAB)])
        plsc.subcore_barrier()

        def _stage2(b, carry):
            base = sid * PER_TILE + b * K
            pltpu.sync_copy(nidx_hbm.at[pl.ds(base, K)], nidx_v)
            pltpu.sync_copy(eidx_hbm.at[pl.ds(base, K)], eidx_v)
            pltpu.async_copy(e_acc.at[eidx_v], rows_v, sem).wait()
            pltpu.sync_copy(rows_v, n_acc.at[nidx_v], add=True)
            return carry

        lax.fori_loop(0, NB, _stage2, 0)
        plsc.subcore_barrier()

        def _final(j, carry):
            nb = sid * N_SLAB + j * CHUNK
            pltpu.sync_copy(n_acc.at[pl.ds(nb, CHUNK)], wbuf)
            pltpu.sync_copy(n_cnt.at[pl.ds(nb, CHUNK)], cbuf)

            def _row(r, c2):
                cnt = cbuf[r, :]
                rs = jnp.where(cnt > 0.0, 1.0 / cnt, 0.0)
                for c in range(Q // 16):
                    v = (
                        wbuf[r, pl.ds(c * 16, 16)] * rs
                        + bias_v[pl.ds(rnd * Q + c * 16, 16)]
                    )
                    obuf[pl.ds(r * Q + c * 16, 16)] = jnp.maximum(v, 0.01 * v)
                return c2

            lax.fori_loop(0, CHUNK, _row, 0)
            pltpu.sync_copy(
                obuf, out_hbm.at[pl.ds((q * N_NODES + nb) * Q, CHUNK * Q)]
            )
            return carry

        lax.fori_loop(0, N_SLAB // CHUNK, _final, 0)
        if rnd == 0:
            plsc.subcore_barrier()


@jax.jit
def kernel(x, hyperedge_index, bias):
    x4 = x.reshape(4 * N_NODES, Q)
    out4 = _hyper_gcn_sc(x4, hyperedge_index[0], hyperedge_index[1], bias)
    out4 = out4.reshape(4, N_NODES, Q).transpose(1, 0, 2)
    return out4.reshape(N_NODES, D_FEAT)


# speedup vs baseline: 12.1779x; 2.4162x over previous
import functools

import jax
import jax.numpy as jnp
from jax import lax
from jax.experimental import pallas as pl
from jax.experimental.pallas import tpu as pltpu
from jax.experimental.pallas import tpu_sc as plsc

N_NODES = 10000
N_EDGES = 2000
D_FEAT = 256
N_INC = 160000
Q = 64
NS = 16
PER_TILE = N_INC // NS
K = 80
NB = PER_TILE // K
E_SLAB = N_EDGES // NS
N_SLAB = N_NODES // NS
CHUNK = 125

_mesh = plsc.VectorSubcoreMesh(
    core_axis_name="c", subcore_axis_name="s", num_cores=2, num_subcores=NS
)


@functools.partial(
    pl.kernel,
    out_type=jax.ShapeDtypeStruct((4 * N_NODES * Q,), jnp.float32),
    mesh=_mesh,
    compiler_params=pltpu.CompilerParams(use_tc_tiling_on_sc=False),
    scratch_types=[
        pltpu.VMEM((2, K, Q), jnp.float32),
        pltpu.VMEM((3, K), jnp.int32),
        pltpu.VMEM((3, K), jnp.int32),
        pltpu.VMEM((3, K), jnp.int32),
        pltpu.VMEM((K, 16), jnp.float32),
        pltpu.VMEM((CHUNK, Q), jnp.float32),
        pltpu.VMEM((CHUNK, 16), jnp.float32),
        pltpu.VMEM((CHUNK * Q,), jnp.float32),
        pltpu.VMEM((2 * Q,), jnp.float32),
        pltpu.VMEM_SHARED((N_EDGES, Q), jnp.float32),
        pltpu.VMEM_SHARED((N_NODES, Q), jnp.float32),
        pltpu.VMEM_SHARED((N_EDGES, 16), jnp.float32),
        pltpu.VMEM_SHARED((N_NODES, 16), jnp.float32),
        pltpu.SemaphoreType.DMA((3,)),
        pltpu.SemaphoreType.DMA((3,)),
        pltpu.SemaphoreType.DMA((3,)),
        pltpu.SemaphoreType.DMA((2,)),
    ],
)
def _hyper_gcn_sc(
    x4, nidx_hbm, eidx_hbm, bias_hbm, out_hbm,
    rows2, nidx2, eidx2, nadj, ones_v, wbuf, cbuf, obuf, bias_v,
    e_acc, n_acc, e_cnt, n_cnt, sem_ni, sem_ei, sem_c, sem_g,
):
    cid = lax.axis_index("c")
    sid = lax.axis_index("s")
    zeros16 = jnp.zeros((16,), jnp.float32)
    ones16 = jnp.ones((16,), jnp.float32)
    ebase = sid * E_SLAB
    tbase = sid * PER_TILE

    def issue_idx(b, s):
        base = tbase + b * K
        pltpu.async_copy(nidx_hbm.at[pl.ds(base, K)], nidx2.at[s], sem_ni.at[s])
        pltpu.async_copy(eidx_hbm.at[pl.ds(base, K)], eidx2.at[s], sem_ei.at[s])

    def wait_idx(s):
        pltpu.make_async_copy(
            nidx_hbm.at[pl.ds(0, K)], nidx2.at[s], sem_ni.at[s]
        ).wait()
        pltpu.make_async_copy(
            eidx_hbm.at[pl.ds(0, K)], eidx2.at[s], sem_ei.at[s]
        ).wait()

    def adjust(s, q):
        for i in range(K // 16):
            nadj[s, pl.ds(i * 16, 16)] = nidx2[s, pl.ds(i * 16, 16)] * 4 + q

    def issue_counts(s):
        pltpu.async_copy(ones_v, n_cnt.at[nidx2.at[s]], sem_c.at[s], add=True)
        pltpu.async_copy(ones_v, e_cnt.at[eidx2.at[s]], sem_c.at[s], add=True)

    def wait_counts(s):
        pltpu.make_async_copy(ones_v, n_cnt.at[nidx2.at[s]], sem_c.at[s]).wait()
        pltpu.make_async_copy(ones_v, e_cnt.at[eidx2.at[s]], sem_c.at[s]).wait()

    def _stage_loop(stage, rnd, q):
        counts = stage == 1 and rnd == 0

        def issue_gather(s, g):
            if stage == 1:
                pltpu.async_copy(x4.at[nadj.at[s]], rows2.at[g], sem_g.at[g])
            else:
                pltpu.async_copy(e_acc.at[eidx2.at[s]], rows2.at[g], sem_g.at[g])

        def wait_gather(s, g):
            if stage == 1:
                pltpu.make_async_copy(
                    x4.at[nadj.at[s]], rows2.at[g], sem_g.at[g]
                ).wait()
            else:
                pltpu.make_async_copy(
                    e_acc.at[eidx2.at[s]], rows2.at[g], sem_g.at[g]
                ).wait()

        issue_idx(0, 0)
        issue_idx(1, 1)
        wait_idx(0)
        if stage == 1:
            adjust(0, q)
        if counts:
            issue_counts(0)
        issue_gather(0, 0)

        def body(b, carry):
            p = lax.rem(b, 3)
            p1 = lax.rem(b + 1, 3)
            p2 = lax.rem(b + 2, 3)
            g = lax.rem(b, 2)
            g1 = lax.rem(b + 1, 2)

            @pl.when(b + 2 < NB)
            def _():
                if counts:
                    @pl.when(b >= 1)
                    def _():
                        wait_counts(p2)
                issue_idx(b + 2, p2)

            @pl.when(b + 1 < NB)
            def _():
                wait_idx(p1)
                if stage == 1:
                    adjust(p1, q)
                if counts:
                    issue_counts(p1)
                issue_gather(p1, g1)

            wait_gather(p, g)
            if stage == 1:
                pltpu.sync_copy(rows2.at[g], e_acc.at[eidx2.at[p]], add=True)
            else:
                pltpu.sync_copy(rows2.at[g], n_acc.at[nidx2.at[p]], add=True)
            return carry

        lax.fori_loop(0, NB, body, 0)
        if counts:
            for b in (NB - 3, NB - 2, NB - 1):
                wait_counts(b % 3)

    def _fill_ones(r, carry):
        ones_v[r, :] = ones16
        return carry

    lax.fori_loop(0, K, _fill_ones, 0)

    def _zero_cnt_row(r, carry):
        cbuf[r, :] = zeros16
        return carry

    lax.fori_loop(0, CHUNK, _zero_cnt_row, 0)

    pltpu.sync_copy(bias_hbm.at[pl.ds(cid * 2 * Q, 2 * Q)], bias_v)

    for rnd in range(2):
        q = 2 * cid + rnd

        def _zero_row(r, carry):
            for c in range(Q // 16):
                wbuf[r, pl.ds(c * 16, 16)] = zeros16
            return carry

        lax.fori_loop(0, CHUNK, _zero_row, 0)
        pltpu.sync_copy(wbuf, e_acc.at[pl.ds(ebase, CHUNK)])
        if rnd == 0:
            pltpu.sync_copy(cbuf, e_cnt.at[pl.ds(ebase, CHUNK)])

        def _zero_nodes(j, carry):
            nb = sid * N_SLAB + j * CHUNK
            pltpu.sync_copy(wbuf, n_acc.at[pl.ds(nb, CHUNK)])
            if rnd == 0:
                pltpu.sync_copy(cbuf, n_cnt.at[pl.ds(nb, CHUNK)])
            return carry

        lax.fori_loop(0, N_SLAB // CHUNK, _zero_nodes, 0)
        plsc.subcore_barrier()

        _stage_loop(1, rnd, q)
        plsc.subcore_barrier()

        pltpu.sync_copy(e_acc.at[pl.ds(ebase, E_SLAB)], wbuf.at[pl.ds(0, E_SLAB)])
        pltpu.sync_copy(e_cnt.at[pl.ds(ebase, E_SLAB)], cbuf.at[pl.ds(0, E_SLAB)])

        def _scale_edge(r, carry):
            cnt = cbuf[r, :]
            rs = jnp.where(cnt > 0.0, 1.0 / cnt, 0.0)
            for c in range(Q // 16):
                wbuf[r, pl.ds(c * 16, 16)] = wbuf[r, pl.ds(c * 16, 16)] * rs
            return carry

        lax.fori_loop(0, E_SLAB, _scale_edge, 0)
        pltpu.sync_copy(wbuf.at[pl.ds(0, E_SLAB)], e_acc.at[pl.ds(ebase, E_SLAB)])
        plsc.subcore_barrier()

        _stage_loop(2, rnd, q)
        plsc.subcore_barrier()

        def _final(j, carry):
            nb = sid * N_SLAB + j * CHUNK
            pltpu.sync_copy(n_acc.at[pl.ds(nb, CHUNK)], wbuf)
            pltpu.sync_copy(n_cnt.at[pl.ds(nb, CHUNK)], cbuf)

            def _row(r, c2):
                cnt = cbuf[r, :]
                rs = jnp.where(cnt > 0.0, 1.0 / cnt, 0.0)
                for c in range(Q // 16):
                    v = (
                        wbuf[r, pl.ds(c * 16, 16)] * rs
                        + bias_v[pl.ds(rnd * Q + c * 16, 16)]
                    )
                    obuf[pl.ds(r * Q + c * 16, 16)] = jnp.maximum(v, 0.01 * v)
                return c2

            lax.fori_loop(0, CHUNK, _row, 0)
            pltpu.sync_copy(
                obuf, out_hbm.at[pl.ds((q * N_NODES + nb) * Q, CHUNK * Q)]
            )
            return carry

        lax.fori_loop(0, N_SLAB // CHUNK, _final, 0)
        if rnd == 0:
            plsc.subcore_barrier()


@jax.jit
def kernel(x, hyperedge_index, bias):
    x4 = x.reshape(4 * N_NODES, Q)
    out4 = _hyper_gcn_sc(x4, hyperedge_index[0], hyperedge_index[1], bias)
    out4 = out4.reshape(4, N_NODES, Q).transpose(1, 0, 2)
    return out4.reshape(N_NODES, D_FEAT)


# speedup vs baseline: 12.2117x; 1.0028x over previous
import functools

import jax
import jax.numpy as jnp
from jax import lax
from jax.experimental import pallas as pl
from jax.experimental.pallas import tpu as pltpu
from jax.experimental.pallas import tpu_sc as plsc

N_NODES = 10000
N_EDGES = 2000
D_FEAT = 256
N_INC = 160000
Q = 64
NS = 16
PER_TILE = N_INC // NS
K = 80
NB = PER_TILE // K
E_SLAB = N_EDGES // NS
N_SLAB = N_NODES // NS
CHUNK = 125

_mesh = plsc.VectorSubcoreMesh(
    core_axis_name="c", subcore_axis_name="s", num_cores=2, num_subcores=NS
)


@functools.partial(
    pl.kernel,
    out_type=jax.ShapeDtypeStruct((4 * N_NODES * Q,), jnp.float32),
    mesh=_mesh,
    compiler_params=pltpu.CompilerParams(use_tc_tiling_on_sc=False),
    scratch_types=[
        pltpu.VMEM((2, K, Q), jnp.float32),
        pltpu.VMEM((3, K), jnp.int32),
        pltpu.VMEM((3, K), jnp.int32),
        pltpu.VMEM((3, K), jnp.int32),
        pltpu.VMEM((K, 16), jnp.float32),
        pltpu.VMEM((CHUNK, Q), jnp.float32),
        pltpu.VMEM((CHUNK, 16), jnp.float32),
        pltpu.VMEM((CHUNK * Q,), jnp.float32),
        pltpu.VMEM((2 * Q,), jnp.float32),
        pltpu.VMEM_SHARED((N_EDGES, Q), jnp.float32),
        pltpu.VMEM_SHARED((N_NODES, Q), jnp.float32),
        pltpu.VMEM_SHARED((N_EDGES, 16), jnp.float32),
        pltpu.VMEM_SHARED((N_NODES, 16), jnp.float32),
        pltpu.SemaphoreType.DMA((3,)),
        pltpu.SemaphoreType.DMA((3,)),
        pltpu.SemaphoreType.DMA((3,)),
        pltpu.SemaphoreType.DMA((2,)),
        pltpu.SemaphoreType.DMA((2,)),
    ],
)
def _hyper_gcn_sc(
    x4, nidx_hbm, eidx_hbm, bias_hbm, out_hbm,
    rows2, nidx2, eidx2, nadj, ones_v, wbuf, cbuf, obuf, bias_v,
    e_acc, n_acc, e_cnt, n_cnt, sem_ni, sem_ei, sem_c, sem_g, sem_s,
):
    cid = lax.axis_index("c")
    sid = lax.axis_index("s")
    zeros16 = jnp.zeros((16,), jnp.float32)
    ones16 = jnp.ones((16,), jnp.float32)
    ebase = sid * E_SLAB
    tbase = sid * PER_TILE

    def issue_idx(b, s):
        base = tbase + b * K
        pltpu.async_copy(nidx_hbm.at[pl.ds(base, K)], nidx2.at[s], sem_ni.at[s])
        pltpu.async_copy(eidx_hbm.at[pl.ds(base, K)], eidx2.at[s], sem_ei.at[s])

    def wait_idx(s):
        pltpu.make_async_copy(
            nidx_hbm.at[pl.ds(0, K)], nidx2.at[s], sem_ni.at[s]
        ).wait()
        pltpu.make_async_copy(
            eidx_hbm.at[pl.ds(0, K)], eidx2.at[s], sem_ei.at[s]
        ).wait()

    def adjust(s, q):
        for i in range(K // 16):
            nadj[s, pl.ds(i * 16, 16)] = nidx2[s, pl.ds(i * 16, 16)] * 4 + q

    def issue_counts(s):
        pltpu.async_copy(ones_v, n_cnt.at[nidx2.at[s]], sem_c.at[s], add=True)
        pltpu.async_copy(ones_v, e_cnt.at[eidx2.at[s]], sem_c.at[s], add=True)

    def wait_counts(s):
        pltpu.make_async_copy(ones_v, n_cnt.at[nidx2.at[s]], sem_c.at[s]).wait()
        pltpu.make_async_copy(ones_v, e_cnt.at[eidx2.at[s]], sem_c.at[s]).wait()

    def _stage_loop(stage, rnd, q):
        counts = stage == 1 and rnd == 0

        def issue_gather(s, g):
            if stage == 1:
                pltpu.async_copy(x4.at[nadj.at[s]], rows2.at[g], sem_g.at[g])
            else:
                pltpu.async_copy(e_acc.at[eidx2.at[s]], rows2.at[g], sem_g.at[g])

        def wait_gather(s, g):
            if stage == 1:
                pltpu.make_async_copy(
                    x4.at[nadj.at[s]], rows2.at[g], sem_g.at[g]
                ).wait()
            else:
                pltpu.make_async_copy(
                    e_acc.at[eidx2.at[s]], rows2.at[g], sem_g.at[g]
                ).wait()

        def issue_scatter(s, g):
            if stage == 1:
                pltpu.async_copy(
                    rows2.at[g], e_acc.at[eidx2.at[s]], sem_s.at[g], add=True
                )
            else:
                pltpu.async_copy(
                    rows2.at[g], n_acc.at[nidx2.at[s]], sem_s.at[g], add=True
                )

        def wait_scatter(s, g):
            if stage == 1:
                pltpu.make_async_copy(
                    rows2.at[g], e_acc.at[eidx2.at[s]], sem_s.at[g]
                ).wait()
            else:
                pltpu.make_async_copy(
                    rows2.at[g], n_acc.at[nidx2.at[s]], sem_s.at[g]
                ).wait()

        issue_idx(0, 0)
        issue_idx(1, 1)
        wait_idx(0)
        if stage == 1:
            adjust(0, q)
        if counts:
            issue_counts(0)
        issue_gather(0, 0)

        def body(b, carry):
            p = lax.rem(b, 3)
            p1 = lax.rem(b + 1, 3)
            p2 = lax.rem(b + 2, 3)
            g = lax.rem(b, 2)
            g1 = lax.rem(b + 1, 2)

            @pl.when(b >= 1)
            def _():
                wait_scatter(p2, g1)

            @pl.when(b + 2 < NB)
            def _():
                if counts:
                    @pl.when(b >= 1)
                    def _():
                        wait_counts(p2)
                issue_idx(b + 2, p2)

            @pl.when(b + 1 < NB)
            def _():
                wait_idx(p1)
                if stage == 1:
                    adjust(p1, q)
                if counts:
                    issue_counts(p1)
                issue_gather(p1, g1)

            wait_gather(p, g)
            issue_scatter(p, g)
            return carry

        lax.fori_loop(0, NB, body, 0)
        wait_scatter((NB - 1) % 3, (NB - 1) % 2)
        if counts:
            for b in (NB - 3, NB - 2, NB - 1):
                wait_counts(b % 3)

    def _fill_ones(r, carry):
        ones_v[r, :] = ones16
        return carry

    lax.fori_loop(0, K, _fill_ones, 0)

    def _zero_cnt_row(r, carry):
        cbuf[r, :] = zeros16
        return carry

    lax.fori_loop(0, CHUNK, _zero_cnt_row, 0)

    pltpu.sync_copy(bias_hbm.at[pl.ds(cid * 2 * Q, 2 * Q)], bias_v)

    for rnd in range(2):
        q = 2 * cid + rnd

        def _zero_row(r, carry):
            for c in range(Q // 16):
                wbuf[r, pl.ds(c * 16, 16)] = zeros16
            return carry

        lax.fori_loop(0, CHUNK, _zero_row, 0)
        pltpu.sync_copy(wbuf, e_acc.at[pl.ds(ebase, CHUNK)])
        if rnd == 0:
            pltpu.sync_copy(cbuf, e_cnt.at[pl.ds(ebase, CHUNK)])

        def _zero_nodes(j, carry):
            nb = sid * N_SLAB + j * CHUNK
            pltpu.sync_copy(wbuf, n_acc.at[pl.ds(nb, CHUNK)])
            if rnd == 0:
                pltpu.sync_copy(cbuf, n_cnt.at[pl.ds(nb, CHUNK)])
            return carry

        lax.fori_loop(0, N_SLAB // CHUNK, _zero_nodes, 0)
        plsc.subcore_barrier()

        _stage_loop(1, rnd, q)
        plsc.subcore_barrier()

        pltpu.sync_copy(e_acc.at[pl.ds(ebase, E_SLAB)], wbuf.at[pl.ds(0, E_SLAB)])
        pltpu.sync_copy(e_cnt.at[pl.ds(ebase, E_SLAB)], cbuf.at[pl.ds(0, E_SLAB)])

        def _scale_edge(r, carry):
            cnt = cbuf[r, :]
            rs = jnp.where(cnt > 0.0, 1.0 / cnt, 0.0)
            for c in range(Q // 16):
                wbuf[r, pl.ds(c * 16, 16)] = wbuf[r, pl.ds(c * 16, 16)] * rs
            return carry

        lax.fori_loop(0, E_SLAB, _scale_edge, 0)
        pltpu.sync_copy(wbuf.at[pl.ds(0, E_SLAB)], e_acc.at[pl.ds(ebase, E_SLAB)])
        plsc.subcore_barrier()

        _stage_loop(2, rnd, q)
        plsc.subcore_barrier()

        def _final(j, carry):
            nb = sid * N_SLAB + j * CHUNK
            pltpu.sync_copy(n_acc.at[pl.ds(nb, CHUNK)], wbuf)
            pltpu.sync_copy(n_cnt.at[pl.ds(nb, CHUNK)], cbuf)

            def _row(r, c2):
                cnt = cbuf[r, :]
                rs = jnp.where(cnt > 0.0, 1.0 / cnt, 0.0)
                for c in range(Q // 16):
                    v = (
                        wbuf[r, pl.ds(c * 16, 16)] * rs
                        + bias_v[pl.ds(rnd * Q + c * 16, 16)]
                    )
                    obuf[pl.ds(r * Q + c * 16, 16)] = jnp.maximum(v, 0.01 * v)
                return c2

            lax.fori_loop(0, CHUNK, _row, 0)
            pltpu.sync_copy(
                obuf, out_hbm.at[pl.ds((q * N_NODES + nb) * Q, CHUNK * Q)]
            )
            return carry

        lax.fori_loop(0, N_SLAB // CHUNK, _final, 0)
        if rnd == 0:
            plsc.subcore_barrier()


@jax.jit
def kernel(x, hyperedge_index, bias):
    x4 = x.reshape(4 * N_NODES, Q)
    out4 = _hyper_gcn_sc(x4, hyperedge_index[0], hyperedge_index[1], bias)
    out4 = out4.reshape(4, N_NODES, Q).transpose(1, 0, 2)
    return out4.reshape(N_NODES, D_FEAT)
